# baseline (speedup 1.0000x reference)
import jax
import jax.numpy as jnp
from jax import lax
from jax.experimental import pallas as pl
from jax.experimental.pallas import tpu as pltpu
from jax.experimental.pallas import tpu_sc as plsc

N = 10000
E = 160000
D = 256
HALF = 128
NC, NS = 2, 16
NP = 10240
EP = 163840
CHUNK = 128
NSUB = 4
SUB = CHUNK // NSUB
JUNK = N
ROWS_PER_TILE = NP // NS
SC_CHUNKS_H = EP // 2 // NS // CHUNK
NBIG = 2
SC_CHUNKS_PAD_H = SC_CHUNKS_H + NBIG
DEG_CHUNKS = EP // (NC * NS) // CHUNK
BM = 256


def _deg_body(dst_hbm, deg_out, dstv, onesv, zv, deg_sh):
    c = lax.axis_index("c")
    s = lax.axis_index("s")
    wid = c * NS + s
    for i in range(CHUNK // 16):
        onesv[pl.ds(i * 16, 16)] = jnp.ones((16,), jnp.float32)
    for i in range(ROWS_PER_TILE // 16):
        zv[pl.ds(i * 16, 16)] = jnp.zeros((16,), jnp.float32)
    pltpu.sync_copy(zv, deg_sh.at[pl.ds(s * ROWS_PER_TILE, ROWS_PER_TILE)])
    pltpu.sync_copy(dst_hbm.at[wid], dstv)
    plsc.subcore_barrier()

    def body(j, carry):
        pltpu.sync_copy(onesv, deg_sh.at[dstv.at[j]], add=True)
        return carry

    lax.fori_loop(0, DEG_CHUNKS, body, 0)
    plsc.subcore_barrier()
    sl = pl.ds(s * ROWS_PER_TILE, ROWS_PER_TILE)
    pltpu.sync_copy(deg_sh.at[sl], deg_out.at[c, sl])


def _scat_body(g0, g1, src_hbm, dst_hbm, s0_in, s1_in, s0_out, s1_out,
               srcv, dstv, bufs, sems, s_sh):
    c = lax.axis_index("c")
    s = lax.axis_index("s")
    sl = pl.ds(s * ROWS_PER_TILE, ROWS_PER_TILE)

    @pl.when(c == 0)
    def _():
        pltpu.sync_copy(s0_in.at[sl], s_sh.at[sl])

    @pl.when(c == 1)
    def _():
        pltpu.sync_copy(s1_in.at[sl], s_sh.at[sl])

    pltpu.sync_copy(src_hbm.at[s], srcv)
    pltpu.sync_copy(dst_hbm.at[s], dstv)
    plsc.subcore_barrier()

    def run(g):
        def gath(j, b):
            for q in range(NSUB):
                ss = pl.ds(q * SUB, SUB)
                pltpu.async_copy(g.at[srcv.at[j, ss]], bufs.at[b, ss],
                                 sems.at[b, q])

        def gwait(j, b):
            for q in range(NSUB):
                ss = pl.ds(q * SUB, SUB)
                pltpu.make_async_copy(g.at[srcv.at[j, ss]], bufs.at[b, ss],
                                      sems.at[b, q]).wait()

        for b in range(NBIG):
            gath(b, b)

        def body(grp, carry):
            j0 = grp * NBIG
            for b in range(NBIG):
                j = j0 + b
                gwait(j, b)
                pltpu.sync_copy(bufs.at[b], s_sh.at[dstv.at[j]], add=True)
                gath(j + NBIG, b)
            return carry

        lax.fori_loop(0, SC_CHUNKS_H // NBIG, body, 0)
        for b in range(NBIG):
            gwait(SC_CHUNKS_H + b, b)

    @pl.when(c == 0)
    def _():
        run(g0)

    @pl.when(c == 1)
    def _():
        run(g1)

    plsc.subcore_barrier()

    @pl.when(c == 0)
    def _():
        pltpu.sync_copy(s_sh.at[sl], s0_out.at[sl])

    @pl.when(c == 1)
    def _():
        pltpu.sync_copy(s_sh.at[sl], s1_out.at[sl])


def _mm_body(x_ref, w_ref, deg_ref, g0_ref, g1_ref):
    dr = deg_ref[...]
    dinv = lax.rsqrt(dr[0] + dr[1] + 1.0)
    h = lax.dot_general(x_ref[...], w_ref[...],
                        (((1,), (1,)), ((), ())),
                        preferred_element_type=jnp.float32)
    g = h * dinv
    g0_ref[...] = g[:, :HALF]
    g1_ref[...] = g[:, HALF:]


def _epi_body(g0_ref, g1_ref, s0_ref, s1_ref, deg_ref, b_ref, out_ref):
    dr = deg_ref[...]
    dinv = lax.rsqrt(dr[0] + dr[1] + 1.0)
    bv = b_ref[...]
    a0 = dinv * (g0_ref[...] + s0_ref[...]) + bv[:, :HALF]
    a1 = dinv * (g1_ref[...] + s1_ref[...]) + bv[:, HALF:]
    out_ref[:, :HALF] = jnp.maximum(a0, 0.0)
    out_ref[:, HALF:] = jnp.maximum(a1, 0.0)


def kernel(x, edge_index, W, b):
    src = edge_index[0].astype(jnp.int32)
    dst = edge_index[1].astype(jnp.int32)
    pad = EP - E
    src_p = jnp.concatenate([src, jnp.zeros((pad,), jnp.int32)])
    dst_p = jnp.concatenate([dst, jnp.full((pad,), JUNK, jnp.int32)])
    src_h = src_p.reshape(2, NS, SC_CHUNKS_H, CHUNK)
    src_h = jnp.concatenate(
        [src_h, jnp.zeros((2, NS, NBIG, CHUNK), jnp.int32)], axis=2)
    dst_h = dst_p.reshape(2, NS, SC_CHUNKS_H, CHUNK)
    dst_d = dst_p.reshape(NC * NS, DEG_CHUNKS, CHUNK)
    x_p = jnp.pad(x, ((0, NP - N), (0, 0)))
    zinit = jnp.zeros((NP, HALF), jnp.float32)

    mesh = plsc.VectorSubcoreMesh(core_axis_name="c", subcore_axis_name="s")

    deg_call = pl.kernel(
        _deg_body,
        out_type=jax.ShapeDtypeStruct((NC, NP), jnp.float32),
        mesh=mesh,
        scratch_types=[
            pltpu.VMEM((DEG_CHUNKS, CHUNK), jnp.int32),
            pltpu.VMEM((CHUNK,), jnp.float32),
            pltpu.VMEM((ROWS_PER_TILE,), jnp.float32),
            pltpu.VMEM_SHARED((NP,), jnp.float32),
        ],
    )
    deg2 = deg_call(dst_d)
    deg3 = deg2[:, :, None]

    grid = NP // BM
    g0, g1 = pl.pallas_call(
        _mm_body,
        grid=(grid,),
        in_specs=[
            pl.BlockSpec((BM, D), lambda i: (i, 0)),
            pl.BlockSpec((D, D), lambda i: (0, 0)),
            pl.BlockSpec((NC, BM, 1), lambda i: (0, i, 0)),
        ],
        out_specs=[
            pl.BlockSpec((BM, HALF), lambda i: (i, 0)),
            pl.BlockSpec((BM, HALF), lambda i: (i, 0)),
        ],
        out_shape=[
            jax.ShapeDtypeStruct((NP, HALF), jnp.float32),
            jax.ShapeDtypeStruct((NP, HALF), jnp.float32),
        ],
    )(x_p, W, deg3)

    scat_call = pl.kernel(
        _scat_body,
        out_type=(
            jax.ShapeDtypeStruct((NP, HALF), jnp.float32),
            jax.ShapeDtypeStruct((NP, HALF), jnp.float32),
        ),
        mesh=mesh,
        scratch_types=[
            pltpu.VMEM((SC_CHUNKS_PAD_H, CHUNK), jnp.int32),
            pltpu.VMEM((SC_CHUNKS_H, CHUNK), jnp.int32),
            pltpu.VMEM((NBIG, CHUNK, HALF), jnp.float32),
            pltpu.SemaphoreType.DMA((NBIG, NSUB)),
            pltpu.VMEM_SHARED((NP, HALF), jnp.float32),
        ],
    )
    s0a, s1a = scat_call(g0, g1, src_h[0], dst_h[0], zinit, zinit)
    s0, s1 = scat_call(g0, g1, src_h[1], dst_h[1], s0a, s1a)

    out = pl.pallas_call(
        _epi_body,
        grid=(grid,),
        in_specs=[
            pl.BlockSpec((BM, HALF), lambda i: (i, 0)),
            pl.BlockSpec((BM, HALF), lambda i: (i, 0)),
            pl.BlockSpec((BM, HALF), lambda i: (i, 0)),
            pl.BlockSpec((BM, HALF), lambda i: (i, 0)),
            pl.BlockSpec((NC, BM, 1), lambda i: (0, i, 0)),
            pl.BlockSpec((1, D), lambda i: (0, 0)),
        ],
        out_specs=pl.BlockSpec((BM, D), lambda i: (i, 0)),
        out_shape=jax.ShapeDtypeStruct((NP, D), jnp.float32),
    )(g0, g1, s0, s1, deg3, b.reshape(1, D))

    return out[:N]


# speedup vs baseline: 1.7393x; 1.7393x over previous
import jax
import jax.numpy as jnp
from jax import lax
from jax.experimental import pallas as pl
from jax.experimental.pallas import tpu as pltpu
from jax.experimental.pallas import tpu_sc as plsc

N = 10000
E = 160000
D = 256
HALF = 128
NC, NS = 2, 16
NP = 10240
EP = 163840
CHUNK = 128
JUNK = N
ROWS_PER_TILE = NP // NS
SC_CHUNKS = EP // NS // CHUNK
NB = 2
DEG_CHUNKS = EP // (NC * NS) // CHUNK
BM = 256


def _deg_body(dst_hbm, deg_out, dstv, onesv, zv, deg_sh):
    c = lax.axis_index("c")
    s = lax.axis_index("s")
    wid = c * NS + s
    for i in range(CHUNK // 16):
        onesv[pl.ds(i * 16, 16)] = jnp.ones((16,), jnp.float32)
    for i in range(ROWS_PER_TILE // 16):
        zv[pl.ds(i * 16, 16)] = jnp.zeros((16,), jnp.float32)
    pltpu.sync_copy(zv, deg_sh.at[pl.ds(s * ROWS_PER_TILE, ROWS_PER_TILE)])
    pltpu.sync_copy(dst_hbm.at[wid], dstv)
    plsc.subcore_barrier()

    def body(j, carry):
        pltpu.sync_copy(onesv, deg_sh.at[dstv.at[j]], add=True)
        return carry

    lax.fori_loop(0, DEG_CHUNKS, body, 0)
    plsc.subcore_barrier()
    sl = pl.ds(s * ROWS_PER_TILE, ROWS_PER_TILE)
    pltpu.sync_copy(deg_sh.at[sl], deg_out.at[c, sl])


def _scat_body(g0, g1, src_hbm, dst_hbm, s0_in, s1_in, s0_out, s1_out,
               srcv, dstring, bufs, gsem, isems, ssems, s_sh):
    c = lax.axis_index("c")
    s = lax.axis_index("s")
    sl = pl.ds(s * ROWS_PER_TILE, ROWS_PER_TILE)

    @pl.when(c == 0)
    def _():
        pltpu.sync_copy(s0_in.at[sl], s_sh.at[sl])

    @pl.when(c == 1)
    def _():
        pltpu.sync_copy(s1_in.at[sl], s_sh.at[sl])

    pltpu.sync_copy(src_hbm.at[s], srcv)
    plsc.subcore_barrier()

    def run(g):
        def step(j, b, drain_prev):
            if drain_prev:
                pltpu.make_async_copy(
                    bufs.at[b], s_sh.at[dstring.at[b]], ssems.at[b]).wait()
            pltpu.async_copy(dst_hbm.at[s, j], dstring.at[b], isems.at[b])
            pltpu.async_copy(g.at[srcv.at[j]], bufs.at[b], gsem).wait()
            pltpu.make_async_copy(
                dst_hbm.at[s, j], dstring.at[b], isems.at[b]).wait()
            pltpu.async_copy(bufs.at[b], s_sh.at[dstring.at[b]], ssems.at[b],
                             add=True)

        for b in range(NB):
            step(b, b, False)

        def body(grp, carry):
            j0 = grp * NB
            for b in range(NB):
                step(j0 + b, b, True)
            return carry

        lax.fori_loop(1, SC_CHUNKS // NB, body, 0)
        for b in range(NB):
            pltpu.make_async_copy(
                bufs.at[b], s_sh.at[dstring.at[b]], ssems.at[b]).wait()

    @pl.when(c == 0)
    def _():
        run(g0)

    @pl.when(c == 1)
    def _():
        run(g1)

    plsc.subcore_barrier()

    @pl.when(c == 0)
    def _():
        pltpu.sync_copy(s_sh.at[sl], s0_out.at[sl])

    @pl.when(c == 1)
    def _():
        pltpu.sync_copy(s_sh.at[sl], s1_out.at[sl])


def _mm_body(x_ref, w_ref, deg_ref, g0_ref, g1_ref):
    dr = deg_ref[...]
    dinv = lax.rsqrt(dr[0] + dr[1] + 1.0)
    h = lax.dot_general(x_ref[...], w_ref[...],
                        (((1,), (1,)), ((), ())),
                        preferred_element_type=jnp.float32)
    g = h * dinv
    g0_ref[...] = g[:, :HALF]
    g1_ref[...] = g[:, HALF:]


def _epi_body(g0_ref, g1_ref, s0_ref, s1_ref, deg_ref, b_ref, out_ref):
    dr = deg_ref[...]
    dinv = lax.rsqrt(dr[0] + dr[1] + 1.0)
    bv = b_ref[...]
    a0 = dinv * (g0_ref[...] + s0_ref[...]) + bv[:, :HALF]
    a1 = dinv * (g1_ref[...] + s1_ref[...]) + bv[:, HALF:]
    out_ref[:, :HALF] = jnp.maximum(a0, 0.0)
    out_ref[:, HALF:] = jnp.maximum(a1, 0.0)


def kernel(x, edge_index, W, b):
    src = edge_index[0].astype(jnp.int32)
    dst = edge_index[1].astype(jnp.int32)
    pad = EP - E
    src_p = jnp.concatenate([src, jnp.zeros((pad,), jnp.int32)])
    dst_p = jnp.concatenate([dst, jnp.full((pad,), JUNK, jnp.int32)])
    src_t = src_p.reshape(NS, SC_CHUNKS, CHUNK)
    dst_t = dst_p.reshape(NS, SC_CHUNKS, CHUNK)
    dst_d = dst_p.reshape(NC * NS, DEG_CHUNKS, CHUNK)
    x_p = jnp.pad(x, ((0, NP - N), (0, 0)))
    zinit = jnp.zeros((NP, HALF), jnp.float32)

    mesh = plsc.VectorSubcoreMesh(core_axis_name="c", subcore_axis_name="s")

    deg_call = pl.kernel(
        _deg_body,
        out_type=jax.ShapeDtypeStruct((NC, NP), jnp.float32),
        mesh=mesh,
        scratch_types=[
            pltpu.VMEM((DEG_CHUNKS, CHUNK), jnp.int32),
            pltpu.VMEM((CHUNK,), jnp.float32),
            pltpu.VMEM((ROWS_PER_TILE,), jnp.float32),
            pltpu.VMEM_SHARED((NP,), jnp.float32),
        ],
    )
    deg2 = deg_call(dst_d)
    deg3 = deg2[:, :, None]

    grid = NP // BM
    g0, g1 = pl.pallas_call(
        _mm_body,
        grid=(grid,),
        in_specs=[
            pl.BlockSpec((BM, D), lambda i: (i, 0)),
            pl.BlockSpec((D, D), lambda i: (0, 0)),
            pl.BlockSpec((NC, BM, 1), lambda i: (0, i, 0)),
        ],
        out_specs=[
            pl.BlockSpec((BM, HALF), lambda i: (i, 0)),
            pl.BlockSpec((BM, HALF), lambda i: (i, 0)),
        ],
        out_shape=[
            jax.ShapeDtypeStruct((NP, HALF), jnp.float32),
            jax.ShapeDtypeStruct((NP, HALF), jnp.float32),
        ],
    )(x_p, W, deg3)

    scat_call = pl.kernel(
        _scat_body,
        out_type=(
            jax.ShapeDtypeStruct((NP, HALF), jnp.float32),
            jax.ShapeDtypeStruct((NP, HALF), jnp.float32),
        ),
        mesh=mesh,
        scratch_types=[
            pltpu.VMEM((SC_CHUNKS, CHUNK), jnp.int32),
            pltpu.VMEM((NB, CHUNK), jnp.int32),
            pltpu.VMEM((NB, CHUNK, HALF), jnp.float32),
            pltpu.SemaphoreType.DMA,
            pltpu.SemaphoreType.DMA((NB,)),
            pltpu.SemaphoreType.DMA((NB,)),
            pltpu.VMEM_SHARED((NP, HALF), jnp.float32),
        ],
    )
    s0, s1 = scat_call(g0, g1, src_t, dst_t, zinit, zinit)

    out = pl.pallas_call(
        _epi_body,
        grid=(grid,),
        in_specs=[
            pl.BlockSpec((BM, HALF), lambda i: (i, 0)),
            pl.BlockSpec((BM, HALF), lambda i: (i, 0)),
            pl.BlockSpec((BM, HALF), lambda i: (i, 0)),
            pl.BlockSpec((BM, HALF), lambda i: (i, 0)),
            pl.BlockSpec((NC, BM, 1), lambda i: (0, i, 0)),
            pl.BlockSpec((1, D), lambda i: (0, 0)),
        ],
        out_specs=pl.BlockSpec((BM, D), lambda i: (i, 0)),
        out_shape=jax.ShapeDtypeStruct((NP, D), jnp.float32),
    )(g0, g1, s0, s1, deg3, b.reshape(1, D))

    return out[:N]


# speedup vs baseline: 1.9385x; 1.1145x over previous
import jax
import jax.numpy as jnp
from jax import lax
from jax.experimental import pallas as pl
from jax.experimental.pallas import tpu as pltpu
from jax.experimental.pallas import tpu_sc as plsc

N = 10000
E = 160000
D = 256
HALF = 128
NC, NS = 2, 16
NP = 10240
EP = 163840
CHUNK = 128
JUNK = N
ROWS_PER_TILE = NP // NS
SC_CHUNKS = EP // NS // CHUNK
NB = 2
DEG_CHUNKS = EP // (NC * NS) // CHUNK
BM = 512
BME = 400


def _deg_body(dst_hbm, deg_out, dstv, onesv, zv, deg_sh):
    c = lax.axis_index("c")
    s = lax.axis_index("s")
    wid = c * NS + s
    for i in range(CHUNK // 16):
        onesv[pl.ds(i * 16, 16)] = jnp.ones((16,), jnp.float32)
    for i in range(ROWS_PER_TILE // 16):
        zv[pl.ds(i * 16, 16)] = jnp.zeros((16,), jnp.float32)
    pltpu.sync_copy(zv, deg_sh.at[pl.ds(s * ROWS_PER_TILE, ROWS_PER_TILE)])
    pltpu.sync_copy(dst_hbm.at[wid], dstv)
    plsc.subcore_barrier()

    def body(j, carry):
        pltpu.sync_copy(onesv, deg_sh.at[dstv.at[j]], add=True)
        return carry

    lax.fori_loop(0, DEG_CHUNKS, body, 0)
    plsc.subcore_barrier()
    sl = pl.ds(s * ROWS_PER_TILE, ROWS_PER_TILE)
    pltpu.sync_copy(deg_sh.at[sl], deg_out.at[c, sl])


def _scat_body(g0, g1, src_hbm, dst_hbm, s0_in, s1_in, s0_out, s1_out,
               srcv, dstring, bufs, gsem, isems, ssems, s_sh):
    c = lax.axis_index("c")
    s = lax.axis_index("s")
    sl = pl.ds(s * ROWS_PER_TILE, ROWS_PER_TILE)

    @pl.when(c == 0)
    def _():
        pltpu.sync_copy(s0_in.at[sl], s_sh.at[sl])

    @pl.when(c == 1)
    def _():
        pltpu.sync_copy(s1_in.at[sl], s_sh.at[sl])

    pltpu.sync_copy(src_hbm.at[s], srcv)
    plsc.subcore_barrier()

    def run(g):
        def step(j, b, drain_prev):
            if drain_prev:
                pltpu.make_async_copy(
                    bufs.at[b], s_sh.at[dstring.at[b]], ssems.at[b]).wait()
            pltpu.async_copy(dst_hbm.at[s, j], dstring.at[b], isems.at[b])
            pltpu.async_copy(g.at[srcv.at[j]], bufs.at[b], gsem).wait()
            pltpu.make_async_copy(
                dst_hbm.at[s, j], dstring.at[b], isems.at[b]).wait()
            pltpu.async_copy(bufs.at[b], s_sh.at[dstring.at[b]], ssems.at[b],
                             add=True)

        for b in range(NB):
            step(b, b, False)

        def body(grp, carry):
            j0 = grp * NB
            for b in range(NB):
                step(j0 + b, b, True)
            return carry

        lax.fori_loop(1, SC_CHUNKS // NB, body, 0)
        for b in range(NB):
            pltpu.make_async_copy(
                bufs.at[b], s_sh.at[dstring.at[b]], ssems.at[b]).wait()

    @pl.when(c == 0)
    def _():
        run(g0)

    @pl.when(c == 1)
    def _():
        run(g1)

    plsc.subcore_barrier()

    @pl.when(c == 0)
    def _():
        pltpu.sync_copy(s_sh.at[sl], s0_out.at[sl])

    @pl.when(c == 1)
    def _():
        pltpu.sync_copy(s_sh.at[sl], s1_out.at[sl])


def _mm_body(x_ref, w_ref, deg_ref, g0_ref, g1_ref):
    dr = deg_ref[...]
    dinv = lax.rsqrt(dr[0] + dr[1] + 1.0)
    h = lax.dot_general(x_ref[...], w_ref[...],
                        (((1,), (1,)), ((), ())),
                        preferred_element_type=jnp.float32)
    g = h * dinv
    g0_ref[...] = g[:, :HALF]
    g1_ref[...] = g[:, HALF:]


def _epi_body(g0_ref, g1_ref, s0_ref, s1_ref, deg_ref, b_ref, out_ref):
    dr = deg_ref[...]
    dinv = lax.rsqrt(dr[0] + dr[1] + 1.0)
    bv = b_ref[...]
    a0 = dinv * (g0_ref[...] + s0_ref[...]) + bv[:, :HALF]
    a1 = dinv * (g1_ref[...] + s1_ref[...]) + bv[:, HALF:]
    out_ref[:, :HALF] = jnp.maximum(a0, 0.0)
    out_ref[:, HALF:] = jnp.maximum(a1, 0.0)


def kernel(x, edge_index, W, b):
    src = edge_index[0].astype(jnp.int32)
    dst = edge_index[1].astype(jnp.int32)
    pad = EP - E
    src_p = jnp.concatenate([src, jnp.zeros((pad,), jnp.int32)])
    dst_p = jnp.concatenate([dst, jnp.full((pad,), JUNK, jnp.int32)])
    src_t = src_p.reshape(NS, SC_CHUNKS, CHUNK)
    dst_t = dst_p.reshape(NS, SC_CHUNKS, CHUNK)
    dst_d = dst_p.reshape(NC * NS, DEG_CHUNKS, CHUNK)
    x_p = jnp.pad(x, ((0, NP - N), (0, 0)))
    zinit = jnp.zeros((NP, HALF), jnp.float32)

    mesh = plsc.VectorSubcoreMesh(core_axis_name="c", subcore_axis_name="s")

    deg_call = pl.kernel(
        _deg_body,
        out_type=jax.ShapeDtypeStruct((NC, NP), jnp.float32),
        mesh=mesh,
        scratch_types=[
            pltpu.VMEM((DEG_CHUNKS, CHUNK), jnp.int32),
            pltpu.VMEM((CHUNK,), jnp.float32),
            pltpu.VMEM((ROWS_PER_TILE,), jnp.float32),
            pltpu.VMEM_SHARED((NP,), jnp.float32),
        ],
    )
    deg2 = deg_call(dst_d)
    deg3 = deg2[:, :, None]

    g0, g1 = pl.pallas_call(
        _mm_body,
        in_specs=[
            pl.BlockSpec((BM, D), lambda i: (i, 0)),
            pl.BlockSpec((D, D), lambda i: (0, 0)),
            pl.BlockSpec((NC, BM, 1), lambda i: (0, i, 0)),
        ],
        out_specs=[
            pl.BlockSpec((BM, HALF), lambda i: (i, 0)),
            pl.BlockSpec((BM, HALF), lambda i: (i, 0)),
        ],
        out_shape=[
            jax.ShapeDtypeStruct((NP, HALF), jnp.float32),
            jax.ShapeDtypeStruct((NP, HALF), jnp.float32),
        ],
        grid=(NP // BM,),
    )(x_p, W, deg3)

    scat_call = pl.kernel(
        _scat_body,
        out_type=(
            jax.ShapeDtypeStruct((NP, HALF), jnp.float32),
            jax.ShapeDtypeStruct((NP, HALF), jnp.float32),
        ),
        mesh=mesh,
        scratch_types=[
            pltpu.VMEM((SC_CHUNKS, CHUNK), jnp.int32),
            pltpu.VMEM((NB, CHUNK), jnp.int32),
            pltpu.VMEM((NB, CHUNK, HALF), jnp.float32),
            pltpu.SemaphoreType.DMA,
            pltpu.SemaphoreType.DMA((NB,)),
            pltpu.SemaphoreType.DMA((NB,)),
            pltpu.VMEM_SHARED((NP, HALF), jnp.float32),
        ],
    )
    s0, s1 = scat_call(g0, g1, src_t, dst_t, zinit, zinit)

    out = pl.pallas_call(
        _epi_body,
        grid=(N // BME,),
        in_specs=[
            pl.BlockSpec((BME, HALF), lambda i: (i, 0)),
            pl.BlockSpec((BME, HALF), lambda i: (i, 0)),
            pl.BlockSpec((BME, HALF), lambda i: (i, 0)),
            pl.BlockSpec((BME, HALF), lambda i: (i, 0)),
            pl.BlockSpec((NC, BME, 1), lambda i: (0, i, 0)),
            pl.BlockSpec((1, D), lambda i: (0, 0)),
        ],
        out_specs=pl.BlockSpec((BME, D), lambda i: (i, 0)),
        out_shape=jax.ShapeDtypeStruct((N, D), jnp.float32),
    )(g0, g1, s0, s1, deg3, b.reshape(1, D))

    return out
